# baseline (speedup 1.0000x reference)
import functools

import jax
import jax.numpy as jnp
from jax import lax
from jax.experimental import pallas as pl
from jax.experimental.pallas import tpu as pltpu
from jax.experimental.pallas import tpu_sc as plsc

N_POINTS = 20000
NUM_CLASSES = 20
C = NUM_CLASSES + 1

NC = 2
NS = 16
L = 16
NW = NC * NS

PT = 640
STEP = 624
G = PT // L
NMAIN = 19968
NTAIL = N_POINTS - NMAIN


def _argmax_tree(vals):
    pairs = [(v, j) for j, v in enumerate(vals)]
    while len(pairs) > 1:
        nxt = []
        for i in range(0, len(pairs) - 1, 2):
            (av, ai), (bv, bi) = pairs[i], pairs[i + 1]
            gt = bv > av
            idx_a = jnp.full((L,), ai, jnp.int32) if isinstance(ai, int) else ai
            idx_b = jnp.full((L,), bi, jnp.int32) if isinstance(bi, int) else bi
            nxt.append((jnp.maximum(av, bv), jnp.where(gt, idx_b, idx_a)))
        if len(pairs) % 2:
            nxt.append(pairs[-1])
        pairs = nxt
    mv, mi = pairs[0]
    mi = jnp.full((L,), mi, jnp.int32) if isinstance(mi, int) else mi
    return mv, mi


def _sum_tree(vals):
    while len(vals) > 1:
        nxt = [vals[i] + vals[i + 1] for i in range(0, len(vals) - 1, 2)]
        if len(vals) % 2:
            nxt.append(vals[-1])
        vals = nxt
    return vals[0]


@functools.partial(
    pl.kernel,
    out_type=(
        jax.ShapeDtypeStruct((2, N_POINTS), jnp.float32),
        jax.ShapeDtypeStruct((N_POINTS,), jnp.int32),
        jax.ShapeDtypeStruct((N_POINTS,), jnp.int32),
    ),
    mesh=plsc.VectorSubcoreMesh(core_axis_name="c", subcore_axis_name="s",
                                num_cores=NC, num_subcores=NS),
    compiler_params=pltpu.CompilerParams(needs_layout_passes=False),
    scratch_types=(
        pltpu.VMEM((2, PT), jnp.float32),
        pltpu.VMEM((C, PT), jnp.float32),
        pltpu.VMEM((NUM_CLASSES + 2,), jnp.float32),
        pltpu.VMEM((2, PT), jnp.float32),
        pltpu.VMEM((PT,), jnp.int32),
        pltpu.VMEM((PT,), jnp.int32),
        pltpu.VMEM((C, NTAIL), jnp.float32),
        pltpu.VMEM((2, NTAIL), jnp.float32),
        pltpu.VMEM((2, NTAIL), jnp.float32),
        pltpu.VMEM((NTAIL,), jnp.int32),
        pltpu.VMEM((NTAIL,), jnp.int32),
    )
    + tuple(pltpu.SemaphoreType.DMA for _ in range(3)),
)
def _sc_select(pts_h, sc_h, aux_h, co_h, cl_h, ro_h,
               pts_v, sc_v, aux_v, co_v, cl_v, ro_v,
               tsc_v, tpt_v, tco_v, tcl_v, tro_v, s0, s1, s2):
    wid = lax.axis_index("s") * NC + lax.axis_index("c")
    is_last = wid == NW - 1
    base = pl.multiple_of((STEP * wid) & ~127, 128)

    H1 = 384
    base2 = pl.multiple_of(base + H1, 128)
    d0 = pltpu.async_copy(sc_h.at[:, pl.ds(base, H1)], sc_v.at[:, pl.ds(0, H1)], s0)
    d1 = pltpu.async_copy(pts_h.at[:, pl.ds(base, PT)], pts_v, s1)
    d2 = pltpu.async_copy(aux_h, aux_v, s2)
    d3 = pltpu.async_copy(sc_h.at[:, pl.ds(base2, PT - H1)], sc_v.at[:, pl.ds(H1, PT - H1)], s0)
    d0.wait()
    d1.wait()
    d2.wait()

    wv = plsc.load_gather(aux_v, [jnp.full((L,), NUM_CLASSES, jnp.int32)])
    hv = plsc.load_gather(aux_v, [jnp.full((L,), NUM_CLASSES + 1, jnp.int32)])

    def select(x, y, vals):
        m, am = _argmax_tree(vals)
        s = _sum_tree([jnp.exp(v - m) for v in vals])
        maxprob = 1.0 / s
        amc = jnp.minimum(am, NUM_CLASSES - 1)
        thrv = plsc.load_gather(aux_v, [amc])
        valid = ((x >= 0.0) & (x < wv) & (y >= 0.0) & (y < hv)
                 & (am < NUM_CLASSES))
        res = valid & (maxprob >= thrv)
        return res, amc

    GH = 384 // L

    @plsc.parallel_loop(0, GH, step=1, unroll=2)
    def group(g):
        b16 = g * L
        x = pts_v[0, pl.ds(b16, L)]
        y = pts_v[1, pl.ds(b16, L)]
        vals = [sc_v[j, pl.ds(b16, L)] for j in range(C)]
        res, amc = select(x, y, vals)
        co_v[0, pl.ds(b16, L)] = jnp.where(res, x, -1.0)
        co_v[1, pl.ds(b16, L)] = jnp.where(res, y, -1.0)
        cl_v[pl.ds(b16, L)] = jnp.where(res, amc, -1)
        ro_v[pl.ds(b16, L)] = res.astype(jnp.int32)

    d3.wait()

    @plsc.parallel_loop(GH, G, step=1, unroll=2)
    def group2(g):
        b16 = g * L
        x = pts_v[0, pl.ds(b16, L)]
        y = pts_v[1, pl.ds(b16, L)]
        vals = [sc_v[j, pl.ds(b16, L)] for j in range(C)]
        res, amc = select(x, y, vals)
        co_v[0, pl.ds(b16, L)] = jnp.where(res, x, -1.0)
        co_v[1, pl.ds(b16, L)] = jnp.where(res, y, -1.0)
        cl_v[pl.ds(b16, L)] = jnp.where(res, amc, -1)
        ro_v[pl.ds(b16, L)] = res.astype(jnp.int32)

    o0 = pltpu.async_copy(co_v, co_h.at[:, pl.ds(base, PT)], s0)
    o1 = pltpu.async_copy(cl_v, cl_h.at[pl.ds(base, PT)], s1)
    o2 = pltpu.async_copy(ro_v, ro_h.at[pl.ds(base, PT)], s2)
    o0.wait()
    o1.wait()
    o2.wait()

    @pl.when(is_last)
    def _tail():
        tb = NMAIN
        t0 = pltpu.async_copy(sc_h.at[:, pl.ds(tb, NTAIL)], tsc_v, s0)
        t1 = pltpu.async_copy(pts_h.at[:, pl.ds(tb, NTAIL)], tpt_v, s1)
        t0.wait()
        t1.wait()
        for g in range(NTAIL // L):
            b16 = g * L
            x = tpt_v[0, pl.ds(b16, L)]
            y = tpt_v[1, pl.ds(b16, L)]
            vals = [tsc_v[j, pl.ds(b16, L)] for j in range(C)]
            res, amc = select(x, y, vals)
            tco_v[0, pl.ds(b16, L)] = jnp.where(res, x, -1.0)
            tco_v[1, pl.ds(b16, L)] = jnp.where(res, y, -1.0)
            tcl_v[pl.ds(b16, L)] = jnp.where(res, amc, -1)
            tro_v[pl.ds(b16, L)] = res.astype(jnp.int32)
        t2 = pltpu.async_copy(tco_v, co_h.at[:, pl.ds(tb, NTAIL)], s0)
        t3 = pltpu.async_copy(tcl_v, cl_h.at[pl.ds(tb, NTAIL)], s1)
        t4 = pltpu.async_copy(tro_v, ro_h.at[pl.ds(tb, NTAIL)], s2)
        t2.wait()
        t3.wait()
        t4.wait()


def kernel(points, scores, score_thr, h, w):
    aux = jnp.concatenate([
        score_thr.astype(jnp.float32),
        jnp.asarray(w, jnp.float32)[None],
        jnp.asarray(h, jnp.float32)[None],
    ])
    ct, cl, ro = _sc_select(points.T, scores.T, aux)
    pred_coords = ct.T
    pred_classes = cl
    reserved = ro.astype(bool)
    return pred_coords, pred_classes, reserved


# speedup vs baseline: 1.0923x; 1.0923x over previous
import functools

import jax
import jax.numpy as jnp
from jax import lax
from jax.experimental import pallas as pl
from jax.experimental.pallas import tpu as pltpu
from jax.experimental.pallas import tpu_sc as plsc

N_POINTS = 20000
NUM_CLASSES = 20
C = NUM_CLASSES + 1

NC = 2
NS = 16
L = 16
NW = NC * NS

PT = 640
STEP = 624
G = PT // L
NMAIN = 19968
NTAIL = N_POINTS - NMAIN


def _argmax_tree(vals):
    pairs = [(v, j) for j, v in enumerate(vals)]
    while len(pairs) > 1:
        nxt = []
        for i in range(0, len(pairs) - 1, 2):
            (av, ai), (bv, bi) = pairs[i], pairs[i + 1]
            gt = bv > av
            idx_a = jnp.full((L,), ai, jnp.int32) if isinstance(ai, int) else ai
            idx_b = jnp.full((L,), bi, jnp.int32) if isinstance(bi, int) else bi
            nxt.append((jnp.maximum(av, bv), jnp.where(gt, idx_b, idx_a)))
        if len(pairs) % 2:
            nxt.append(pairs[-1])
        pairs = nxt
    mv, mi = pairs[0]
    mi = jnp.full((L,), mi, jnp.int32) if isinstance(mi, int) else mi
    return mv, mi


def _sum_tree(vals):
    while len(vals) > 1:
        nxt = [vals[i] + vals[i + 1] for i in range(0, len(vals) - 1, 2)]
        if len(vals) % 2:
            nxt.append(vals[-1])
        vals = nxt
    return vals[0]


@functools.partial(
    pl.kernel,
    out_type=(
        jax.ShapeDtypeStruct((2, N_POINTS), jnp.float32),
        jax.ShapeDtypeStruct((N_POINTS,), jnp.int32),
        jax.ShapeDtypeStruct((N_POINTS,), jnp.int32),
    ),
    mesh=plsc.VectorSubcoreMesh(core_axis_name="c", subcore_axis_name="s",
                                num_cores=NC, num_subcores=NS),
    compiler_params=pltpu.CompilerParams(needs_layout_passes=False),
    scratch_types=(
        pltpu.VMEM((2, PT), jnp.float32),
        pltpu.VMEM((C, PT), jnp.float32),
        pltpu.VMEM((NUM_CLASSES + 2,), jnp.float32),
        pltpu.VMEM((2, PT), jnp.float32),
        pltpu.VMEM((PT,), jnp.int32),
        pltpu.VMEM((PT,), jnp.int32),
        pltpu.VMEM((C, NTAIL), jnp.float32),
        pltpu.VMEM((2, NTAIL), jnp.float32),
        pltpu.VMEM((2, NTAIL), jnp.float32),
        pltpu.VMEM((NTAIL,), jnp.int32),
        pltpu.VMEM((NTAIL,), jnp.int32),
    )
    + tuple(pltpu.SemaphoreType.DMA for _ in range(3)),
)
def _sc_select(pts_h, sc_h, aux_h, co_h, cl_h, ro_h,
               pts_v, sc_v, aux_v, co_v, cl_v, ro_v,
               tsc_v, tpt_v, tco_v, tcl_v, tro_v, s0, s1, s2):
    wid = lax.axis_index("s") * NC + lax.axis_index("c")
    is_last = wid == NW - 1
    base = pl.multiple_of((STEP * wid) & ~127, 128)

    d0 = pltpu.async_copy(sc_h.at[:, pl.ds(base, PT)], sc_v, s0)
    d1 = pltpu.async_copy(pts_h.at[:, pl.ds(base, PT)], pts_v, s1)
    d2 = pltpu.async_copy(aux_h, aux_v, s2)
    d0.wait()
    d1.wait()
    d2.wait()

    wv = plsc.load_gather(aux_v, [jnp.full((L,), NUM_CLASSES, jnp.int32)])
    hv = plsc.load_gather(aux_v, [jnp.full((L,), NUM_CLASSES + 1, jnp.int32)])

    def select(x, y, vals):
        m, am = _argmax_tree(vals)
        s = _sum_tree([jnp.exp(v - m) for v in vals])
        maxprob = 1.0 / s
        amc = jnp.minimum(am, NUM_CLASSES - 1)
        thrv = plsc.load_gather(aux_v, [amc])
        valid = ((x >= 0.0) & (x < wv) & (y >= 0.0) & (y < hv)
                 & (am < NUM_CLASSES))
        res = valid & (maxprob >= thrv)
        return res, amc

    @plsc.parallel_loop(0, G, step=1, unroll=1)
    def group(g):
        b16 = g * L
        x = pts_v[0, pl.ds(b16, L)]
        y = pts_v[1, pl.ds(b16, L)]
        vals = [sc_v[j, pl.ds(b16, L)] for j in range(C)]
        res, amc = select(x, y, vals)
        co_v[0, pl.ds(b16, L)] = jnp.where(res, x, -1.0)
        co_v[1, pl.ds(b16, L)] = jnp.where(res, y, -1.0)
        cl_v[pl.ds(b16, L)] = jnp.where(res, amc, -1)
        ro_v[pl.ds(b16, L)] = res.astype(jnp.int32)

    o0 = pltpu.async_copy(co_v, co_h.at[:, pl.ds(base, PT)], s0)
    o1 = pltpu.async_copy(cl_v, cl_h.at[pl.ds(base, PT)], s1)
    o2 = pltpu.async_copy(ro_v, ro_h.at[pl.ds(base, PT)], s2)
    o0.wait()
    o1.wait()
    o2.wait()

    @pl.when(is_last)
    def _tail():
        tb = NMAIN
        t0 = pltpu.async_copy(sc_h.at[:, pl.ds(tb, NTAIL)], tsc_v, s0)
        t1 = pltpu.async_copy(pts_h.at[:, pl.ds(tb, NTAIL)], tpt_v, s1)
        t0.wait()
        t1.wait()
        for g in range(NTAIL // L):
            b16 = g * L
            x = tpt_v[0, pl.ds(b16, L)]
            y = tpt_v[1, pl.ds(b16, L)]
            vals = [tsc_v[j, pl.ds(b16, L)] for j in range(C)]
            res, amc = select(x, y, vals)
            tco_v[0, pl.ds(b16, L)] = jnp.where(res, x, -1.0)
            tco_v[1, pl.ds(b16, L)] = jnp.where(res, y, -1.0)
            tcl_v[pl.ds(b16, L)] = jnp.where(res, amc, -1)
            tro_v[pl.ds(b16, L)] = res.astype(jnp.int32)
        t2 = pltpu.async_copy(tco_v, co_h.at[:, pl.ds(tb, NTAIL)], s0)
        t3 = pltpu.async_copy(tcl_v, cl_h.at[pl.ds(tb, NTAIL)], s1)
        t4 = pltpu.async_copy(tro_v, ro_h.at[pl.ds(tb, NTAIL)], s2)
        t2.wait()
        t3.wait()
        t4.wait()


def kernel(points, scores, score_thr, h, w):
    aux = jnp.concatenate([
        score_thr.astype(jnp.float32),
        jnp.asarray(w, jnp.float32)[None],
        jnp.asarray(h, jnp.float32)[None],
    ])
    ct, cl, ro = _sc_select(points.T, scores.T, aux)
    pred_coords = ct.T
    pred_classes = cl
    reserved = ro.astype(bool)
    return pred_coords, pred_classes, reserved


# speedup vs baseline: 1.1089x; 1.0152x over previous
import functools

import jax
import jax.numpy as jnp
from jax import lax
from jax.experimental import pallas as pl
from jax.experimental.pallas import tpu as pltpu
from jax.experimental.pallas import tpu_sc as plsc

N_POINTS = 20000
NUM_CLASSES = 20
C = NUM_CLASSES + 1

NC = 2
NS = 16
L = 16
NW = NC * NS

PT = 640
STEP = 624
G = PT // L
NMAIN = 19968
NTAIL = N_POINTS - NMAIN


def _argmax_tree(vals):
    pairs = [(v, j) for j, v in enumerate(vals)]
    while len(pairs) > 1:
        nxt = []
        for i in range(0, len(pairs) - 1, 2):
            (av, ai), (bv, bi) = pairs[i], pairs[i + 1]
            gt = bv > av
            idx_a = jnp.full((L,), ai, jnp.int32) if isinstance(ai, int) else ai
            idx_b = jnp.full((L,), bi, jnp.int32) if isinstance(bi, int) else bi
            nxt.append((jnp.maximum(av, bv), jnp.where(gt, idx_b, idx_a)))
        if len(pairs) % 2:
            nxt.append(pairs[-1])
        pairs = nxt
    mv, mi = pairs[0]
    mi = jnp.full((L,), mi, jnp.int32) if isinstance(mi, int) else mi
    return mv, mi


def _sum_tree(vals):
    while len(vals) > 1:
        nxt = [vals[i] + vals[i + 1] for i in range(0, len(vals) - 1, 2)]
        if len(vals) % 2:
            nxt.append(vals[-1])
        vals = nxt
    return vals[0]


@functools.partial(
    pl.kernel,
    out_type=(
        jax.ShapeDtypeStruct((2, N_POINTS), jnp.float32),
        jax.ShapeDtypeStruct((N_POINTS,), jnp.int32),
    ),
    mesh=plsc.VectorSubcoreMesh(core_axis_name="c", subcore_axis_name="s",
                                num_cores=NC, num_subcores=NS),
    compiler_params=pltpu.CompilerParams(needs_layout_passes=False),
    scratch_types=(
        pltpu.VMEM((2, PT), jnp.float32),
        pltpu.VMEM((C, PT), jnp.float32),
        pltpu.VMEM((NUM_CLASSES + 3,), jnp.float32),
        pltpu.VMEM((2, PT), jnp.float32),
        pltpu.VMEM((PT,), jnp.int32),
        pltpu.VMEM((C, NTAIL), jnp.float32),
        pltpu.VMEM((2, NTAIL), jnp.float32),
        pltpu.VMEM((2, NTAIL), jnp.float32),
        pltpu.VMEM((NTAIL,), jnp.int32),
    )
    + tuple(pltpu.SemaphoreType.DMA for _ in range(3)),
)
def _sc_select(pts_h, sc_h, aux_h, co_h, cl_h,
               pts_v, sc_v, aux_v, co_v, cl_v,
               tsc_v, tpt_v, tco_v, tcl_v, s0, s1, s2):
    wid = lax.axis_index("s") * NC + lax.axis_index("c")
    is_last = wid == NW - 1
    base = pl.multiple_of((STEP * wid) & ~127, 128)

    d0 = pltpu.async_copy(sc_h.at[:, pl.ds(base, PT)], sc_v, s0)
    d1 = pltpu.async_copy(pts_h.at[:, pl.ds(base, PT)], pts_v, s1)
    d2 = pltpu.async_copy(aux_h, aux_v, s2)
    d0.wait()
    d1.wait()
    d2.wait()

    wv = plsc.load_gather(aux_v, [jnp.full((L,), NUM_CLASSES + 1, jnp.int32)])
    hv = plsc.load_gather(aux_v, [jnp.full((L,), NUM_CLASSES + 2, jnp.int32)])

    def select(x, y, vals):
        m, am = _argmax_tree(vals)
        s = _sum_tree([jnp.exp(v - m) for v in vals])
        maxprob = 1.0 / s
        thrv = plsc.load_gather(aux_v, [am])
        valid = (x >= 0.0) & (x < wv) & (y >= 0.0) & (y < hv)
        res = valid & (maxprob >= thrv)
        return res, am

    @plsc.parallel_loop(0, G, step=1, unroll=1)
    def group(g):
        b16 = g * L
        x = pts_v[0, pl.ds(b16, L)]
        y = pts_v[1, pl.ds(b16, L)]
        vals = [sc_v[j, pl.ds(b16, L)] for j in range(C)]
        res, am = select(x, y, vals)
        co_v[0, pl.ds(b16, L)] = jnp.where(res, x, -1.0)
        co_v[1, pl.ds(b16, L)] = jnp.where(res, y, -1.0)
        cl_v[pl.ds(b16, L)] = jnp.where(res, am, -1)

    o0 = pltpu.async_copy(co_v, co_h.at[:, pl.ds(base, PT)], s0)
    o1 = pltpu.async_copy(cl_v, cl_h.at[pl.ds(base, PT)], s1)
    o0.wait()
    o1.wait()

    @pl.when(is_last)
    def _tail():
        tb = NMAIN
        t0 = pltpu.async_copy(sc_h.at[:, pl.ds(tb, NTAIL)], tsc_v, s0)
        t1 = pltpu.async_copy(pts_h.at[:, pl.ds(tb, NTAIL)], tpt_v, s1)
        t0.wait()
        t1.wait()
        for g in range(NTAIL // L):
            b16 = g * L
            x = tpt_v[0, pl.ds(b16, L)]
            y = tpt_v[1, pl.ds(b16, L)]
            vals = [tsc_v[j, pl.ds(b16, L)] for j in range(C)]
            res, am = select(x, y, vals)
            tco_v[0, pl.ds(b16, L)] = jnp.where(res, x, -1.0)
            tco_v[1, pl.ds(b16, L)] = jnp.where(res, y, -1.0)
            tcl_v[pl.ds(b16, L)] = jnp.where(res, am, -1)
        t2 = pltpu.async_copy(tco_v, co_h.at[:, pl.ds(tb, NTAIL)], s0)
        t3 = pltpu.async_copy(tcl_v, cl_h.at[pl.ds(tb, NTAIL)], s1)
        t2.wait()
        t3.wait()


def kernel(points, scores, score_thr, h, w):
    aux = jnp.concatenate([
        score_thr.astype(jnp.float32),
        jnp.full((1,), jnp.inf, jnp.float32),
        jnp.asarray(w, jnp.float32)[None],
        jnp.asarray(h, jnp.float32)[None],
    ])
    ct, cl = _sc_select(points.T, scores.T, aux)
    pred_coords = ct.T
    pred_classes = cl
    reserved = cl >= 0
    return pred_coords, pred_classes, reserved


# speedup vs baseline: 1.1101x; 1.0011x over previous
import functools

import jax
import jax.numpy as jnp
from jax import lax
from jax.experimental import pallas as pl
from jax.experimental.pallas import tpu as pltpu
from jax.experimental.pallas import tpu_sc as plsc

N_POINTS = 20000
NUM_CLASSES = 20
C = NUM_CLASSES + 1

NC = 2
NS = 16
L = 16
NW = NC * NS

PT = 640
STEP = 624
G = PT // L
NMAIN = 19968
NTAIL = N_POINTS - NMAIN


def _argmax_tree(vals):
    pairs = [(v, j) for j, v in enumerate(vals)]
    while len(pairs) > 1:
        nxt = []
        for i in range(0, len(pairs) - 1, 2):
            (av, ai), (bv, bi) = pairs[i], pairs[i + 1]
            gt = bv > av
            idx_a = jnp.full((L,), ai, jnp.int32) if isinstance(ai, int) else ai
            idx_b = jnp.full((L,), bi, jnp.int32) if isinstance(bi, int) else bi
            nxt.append((jnp.maximum(av, bv), jnp.where(gt, idx_b, idx_a)))
        if len(pairs) % 2:
            nxt.append(pairs[-1])
        pairs = nxt
    mv, mi = pairs[0]
    mi = jnp.full((L,), mi, jnp.int32) if isinstance(mi, int) else mi
    return mv, mi


def _sum_tree(vals):
    while len(vals) > 1:
        nxt = [vals[i] + vals[i + 1] for i in range(0, len(vals) - 1, 2)]
        if len(vals) % 2:
            nxt.append(vals[-1])
        vals = nxt
    return vals[0]


@functools.partial(
    pl.kernel,
    out_type=(
        jax.ShapeDtypeStruct((2, N_POINTS), jnp.float32),
        jax.ShapeDtypeStruct((N_POINTS,), jnp.int32),
    ),
    mesh=plsc.VectorSubcoreMesh(core_axis_name="c", subcore_axis_name="s",
                                num_cores=NC, num_subcores=NS),
    compiler_params=pltpu.CompilerParams(needs_layout_passes=False,
                                         skip_device_barrier=True),
    scratch_types=(
        pltpu.VMEM((2, PT), jnp.float32),
        pltpu.VMEM((C, PT), jnp.float32),
        pltpu.VMEM((NUM_CLASSES + 3,), jnp.float32),
        pltpu.VMEM((2, PT), jnp.float32),
        pltpu.VMEM((PT,), jnp.int32),
        pltpu.VMEM((C, NTAIL), jnp.float32),
        pltpu.VMEM((2, NTAIL), jnp.float32),
        pltpu.VMEM((2, NTAIL), jnp.float32),
        pltpu.VMEM((NTAIL,), jnp.int32),
    )
    + tuple(pltpu.SemaphoreType.DMA for _ in range(3)),
)
def _sc_select(pts_h, sc_h, aux_h, co_h, cl_h,
               pts_v, sc_v, aux_v, co_v, cl_v,
               tsc_v, tpt_v, tco_v, tcl_v, s0, s1, s2):
    wid = lax.axis_index("s") * NC + lax.axis_index("c")
    is_last = wid == NW - 1
    base = pl.multiple_of((STEP * wid) & ~127, 128)

    d0 = pltpu.async_copy(sc_h.at[:, pl.ds(base, PT)], sc_v, s0)
    d1 = pltpu.async_copy(pts_h.at[:, pl.ds(base, PT)], pts_v, s1)
    d2 = pltpu.async_copy(aux_h, aux_v, s2)
    d0.wait()
    d1.wait()
    d2.wait()

    wv = plsc.load_gather(aux_v, [jnp.full((L,), NUM_CLASSES + 1, jnp.int32)])
    hv = plsc.load_gather(aux_v, [jnp.full((L,), NUM_CLASSES + 2, jnp.int32)])

    def select(x, y, vals):
        m, am = _argmax_tree(vals)
        s = _sum_tree([jnp.exp(v - m) for v in vals])
        maxprob = 1.0 / s
        thrv = plsc.load_gather(aux_v, [am])
        valid = (x >= 0.0) & (x < wv) & (y >= 0.0) & (y < hv)
        res = valid & (maxprob >= thrv)
        return res, am

    @plsc.parallel_loop(0, G, step=1, unroll=1)
    def group(g):
        b16 = g * L
        x = pts_v[0, pl.ds(b16, L)]
        y = pts_v[1, pl.ds(b16, L)]
        vals = [sc_v[j, pl.ds(b16, L)] for j in range(C)]
        res, am = select(x, y, vals)
        co_v[0, pl.ds(b16, L)] = jnp.where(res, x, -1.0)
        co_v[1, pl.ds(b16, L)] = jnp.where(res, y, -1.0)
        cl_v[pl.ds(b16, L)] = jnp.where(res, am, -1)

    o0 = pltpu.async_copy(co_v, co_h.at[:, pl.ds(base, PT)], s0)
    o1 = pltpu.async_copy(cl_v, cl_h.at[pl.ds(base, PT)], s1)
    o0.wait()
    o1.wait()

    @pl.when(is_last)
    def _tail():
        tb = NMAIN
        t0 = pltpu.async_copy(sc_h.at[:, pl.ds(tb, NTAIL)], tsc_v, s0)
        t1 = pltpu.async_copy(pts_h.at[:, pl.ds(tb, NTAIL)], tpt_v, s1)
        t0.wait()
        t1.wait()
        for g in range(NTAIL // L):
            b16 = g * L
            x = tpt_v[0, pl.ds(b16, L)]
            y = tpt_v[1, pl.ds(b16, L)]
            vals = [tsc_v[j, pl.ds(b16, L)] for j in range(C)]
            res, am = select(x, y, vals)
            tco_v[0, pl.ds(b16, L)] = jnp.where(res, x, -1.0)
            tco_v[1, pl.ds(b16, L)] = jnp.where(res, y, -1.0)
            tcl_v[pl.ds(b16, L)] = jnp.where(res, am, -1)
        t2 = pltpu.async_copy(tco_v, co_h.at[:, pl.ds(tb, NTAIL)], s0)
        t3 = pltpu.async_copy(tcl_v, cl_h.at[pl.ds(tb, NTAIL)], s1)
        t2.wait()
        t3.wait()


def kernel(points, scores, score_thr, h, w):
    aux = jnp.concatenate([
        score_thr.astype(jnp.float32),
        jnp.full((1,), jnp.inf, jnp.float32),
        jnp.asarray(w, jnp.float32)[None],
        jnp.asarray(h, jnp.float32)[None],
    ])
    ct, cl = _sc_select(points.T, scores.T, aux)
    pred_coords = ct.T
    pred_classes = cl
    reserved = cl >= 0
    return pred_coords, pred_classes, reserved


# speedup vs baseline: 1.1108x; 1.0006x over previous
import functools

import jax
import jax.numpy as jnp
from jax import lax
from jax.experimental import pallas as pl
from jax.experimental.pallas import tpu as pltpu
from jax.experimental.pallas import tpu_sc as plsc

N_POINTS = 20000
NUM_CLASSES = 20
C = NUM_CLASSES + 1

NC = 2
NS = 16
L = 16
NW = NC * NS

PT = 640
STEP = 624
G = PT // L
NMAIN = 19968
NTAIL = N_POINTS - NMAIN


def _argmax_tree(vals):
    pairs = [(v, j) for j, v in enumerate(vals)]
    while len(pairs) > 1:
        nxt = []
        for i in range(0, len(pairs) - 1, 2):
            (av, ai), (bv, bi) = pairs[i], pairs[i + 1]
            gt = bv > av
            idx_a = jnp.full((L,), ai, jnp.int32) if isinstance(ai, int) else ai
            idx_b = jnp.full((L,), bi, jnp.int32) if isinstance(bi, int) else bi
            nxt.append((jnp.maximum(av, bv), jnp.where(gt, idx_b, idx_a)))
        if len(pairs) % 2:
            nxt.append(pairs[-1])
        pairs = nxt
    mv, mi = pairs[0]
    mi = jnp.full((L,), mi, jnp.int32) if isinstance(mi, int) else mi
    return mv, mi


def _sum_tree(vals):
    while len(vals) > 1:
        nxt = [vals[i] + vals[i + 1] for i in range(0, len(vals) - 1, 2)]
        if len(vals) % 2:
            nxt.append(vals[-1])
        vals = nxt
    return vals[0]


@functools.partial(
    pl.kernel,
    out_type=(
        jax.ShapeDtypeStruct((2, N_POINTS), jnp.float32),
        jax.ShapeDtypeStruct((N_POINTS,), jnp.int32),
    ),
    mesh=plsc.VectorSubcoreMesh(core_axis_name="c", subcore_axis_name="s",
                                num_cores=NC, num_subcores=NS),
    compiler_params=pltpu.CompilerParams(needs_layout_passes=False),
    scratch_types=(
        pltpu.VMEM((2, PT), jnp.float32),
        pltpu.VMEM((C, PT), jnp.float32),
        pltpu.VMEM((NUM_CLASSES + 3,), jnp.float32),
        pltpu.VMEM((2, PT), jnp.float32),
        pltpu.VMEM((PT,), jnp.int32),
        pltpu.VMEM((C, NTAIL), jnp.float32),
        pltpu.VMEM((2, NTAIL), jnp.float32),
        pltpu.VMEM((2, NTAIL), jnp.float32),
        pltpu.VMEM((NTAIL,), jnp.int32),
    )
    + tuple(pltpu.SemaphoreType.DMA for _ in range(3)),
)
def _sc_select(pts_h, sc_h, aux_h, co_h, cl_h,
               pts_v, sc_v, aux_v, co_v, cl_v,
               tsc_v, tpt_v, tco_v, tcl_v, s0, s1, s2):
    wid = lax.axis_index("s") * NC + lax.axis_index("c")
    is_last = wid == NW - 1
    base = pl.multiple_of((STEP * wid) & ~127, 128)

    d0 = pltpu.async_copy(sc_h.at[:, pl.ds(base, PT)], sc_v, s0)
    d1 = pltpu.async_copy(pts_h.at[:, pl.ds(base, PT)], pts_v, s1)
    d2 = pltpu.async_copy(aux_h, aux_v, s2)
    d0.wait()
    d1.wait()
    d2.wait()

    wv = plsc.load_gather(aux_v, [jnp.full((L,), NUM_CLASSES + 1, jnp.int32)])
    hv = plsc.load_gather(aux_v, [jnp.full((L,), NUM_CLASSES + 2, jnp.int32)])

    def select(x, y, vals):
        m, am = _argmax_tree(vals)
        s = _sum_tree([jnp.exp(v - m) for v in vals])
        maxprob = 1.0 / s
        thrv = plsc.load_gather(aux_v, [am])
        valid = (x >= 0.0) & (x < wv) & (y >= 0.0) & (y < hv)
        res = valid & (maxprob >= thrv)
        return res, am

    @plsc.parallel_loop(0, G, step=1, unroll=1)
    def group(g):
        b16 = g * L
        x = pts_v[0, pl.ds(b16, L)]
        y = pts_v[1, pl.ds(b16, L)]
        vals = [sc_v[j, pl.ds(b16, L)] for j in range(C)]
        res, am = select(x, y, vals)
        co_v[0, pl.ds(b16, L)] = jnp.where(res, x, -1.0)
        co_v[1, pl.ds(b16, L)] = jnp.where(res, y, -1.0)
        cl_v[pl.ds(b16, L)] = jnp.where(res, am, -1)

    o0 = pltpu.async_copy(co_v, co_h.at[:, pl.ds(base, PT)], s0)
    o1 = pltpu.async_copy(cl_v, cl_h.at[pl.ds(base, PT)], s1)
    o0.wait()
    o1.wait()

    @pl.when(is_last)
    def _tail():
        tb = NMAIN
        t0 = pltpu.async_copy(sc_h.at[:, pl.ds(tb, NTAIL)], tsc_v, s0)
        t1 = pltpu.async_copy(pts_h.at[:, pl.ds(tb, NTAIL)], tpt_v, s1)
        t0.wait()
        t1.wait()
        for g in range(NTAIL // L):
            b16 = g * L
            x = tpt_v[0, pl.ds(b16, L)]
            y = tpt_v[1, pl.ds(b16, L)]
            vals = [tsc_v[j, pl.ds(b16, L)] for j in range(C)]
            res, am = select(x, y, vals)
            tco_v[0, pl.ds(b16, L)] = jnp.where(res, x, -1.0)
            tco_v[1, pl.ds(b16, L)] = jnp.where(res, y, -1.0)
            tcl_v[pl.ds(b16, L)] = jnp.where(res, am, -1)
        t2 = pltpu.async_copy(tco_v, co_h.at[:, pl.ds(tb, NTAIL)], s0)
        t3 = pltpu.async_copy(tcl_v, cl_h.at[pl.ds(tb, NTAIL)], s1)
        t2.wait()
        t3.wait()


def kernel(points, scores, score_thr, h, w):
    aux = jnp.concatenate([
        score_thr.astype(jnp.float32),
        jnp.full((1,), jnp.inf, jnp.float32),
        jnp.asarray(w, jnp.float32)[None],
        jnp.asarray(h, jnp.float32)[None],
    ])
    ct, cl = _sc_select(points.T, scores.T, aux)
    pred_coords = ct.T
    pred_classes = cl
    reserved = cl >= 0
    return pred_coords, pred_classes, reserved
